# Initial kernel scaffold; baseline (speedup 1.0000x reference)
import functools

import jax
import jax.numpy as jnp
from jax import lax
from jax.experimental import pallas as pl
from jax.experimental.pallas import tpu as pltpu
from jax.experimental.pallas import tpu_sc as plsc

_VOCAB = 1000000
_CLASSES = 1000
_D = 32
_B = 4096
_L = 200

_NC = 2
_NS = 16
_NW = _NC * _NS
_ROWS_PER_W = _B // _NW
_HALF = _L // 2
_NBUF = 4


def _sc_pool_body(x_hbm, table_hbm, out_hbm, idx_v, bufs, acc, sems):
    wid = lax.axis_index("s") * _NC + lax.axis_index("c")
    idx_base = wid * (2 * _ROWS_PER_W)
    row_base = wid * _ROWS_PER_W

    pltpu.sync_copy(x_hbm.at[pl.ds(idx_base, 2 * _ROWS_PER_W)], idx_v)

    def fire(b, s):
        pltpu.async_copy(
            table_hbm.at[idx_v.at[2 * b]],
            bufs.at[s, pl.ds(0, _HALF)],
            sems.at[s],
        )
        pltpu.async_copy(
            table_hbm.at[idx_v.at[2 * b + 1]],
            bufs.at[s, pl.ds(_HALF, _HALF)],
            sems.at[s],
        )

    def drain(b, s):
        pltpu.make_async_copy(
            table_hbm.at[idx_v.at[2 * b]],
            bufs.at[s, pl.ds(0, _HALF)],
            sems.at[s],
        ).wait()
        pltpu.make_async_copy(
            table_hbm.at[idx_v.at[2 * b + 1]],
            bufs.at[s, pl.ds(_HALF, _HALF)],
            sems.at[s],
        ).wait()

    for s in range(_NBUF):
        fire(s, s)

    zeros = jnp.zeros((16,), jnp.float32)

    def outer(bb, carry):
        for s in range(_NBUF):
            b = bb * _NBUF + s
            drain(b, s)

            def body(r, c):
                a0, a1, a2, a3 = c
                a0 = a0 + bufs[s, 2 * r, pl.ds(0, 16)]
                a1 = a1 + bufs[s, 2 * r, pl.ds(16, 16)]
                a2 = a2 + bufs[s, 2 * r + 1, pl.ds(0, 16)]
                a3 = a3 + bufs[s, 2 * r + 1, pl.ds(16, 16)]
                return (a0, a1, a2, a3)

            nb = b + _NBUF

            @pl.when(nb < _ROWS_PER_W)
            def _():
                fire(nb, s)

            a0, a1, a2, a3 = lax.fori_loop(
                0, _L // 2, body, (zeros, zeros, zeros, zeros), unroll=2
            )
            acc[b, pl.ds(0, 16)] = a0 + a2
            acc[b, pl.ds(16, 16)] = a1 + a3
        return carry

    lax.fori_loop(0, _ROWS_PER_W // _NBUF, outer, 0)

    pltpu.sync_copy(acc, out_hbm.at[pl.ds(row_base, _ROWS_PER_W)])


_sc_pool = functools.partial(
    pl.kernel,
    mesh=plsc.VectorSubcoreMesh(core_axis_name="c", subcore_axis_name="s"),
    out_type=jax.ShapeDtypeStruct((_B, _D), jnp.float32),
    scratch_types=[
        pltpu.VMEM((2 * _ROWS_PER_W, _HALF), jnp.int32),
        pltpu.VMEM((_NBUF, _L, _D), jnp.float32),
        pltpu.VMEM((_ROWS_PER_W, _D), jnp.float32),
        pltpu.SemaphoreType.DMA((_NBUF,)),
    ],
)(_sc_pool_body)


def _tc_head_body(p_ref, w_ref, b_ref, o_ref):
    t = jnp.tanh(p_ref[...] * (1.0 / _L))
    o_ref[...] = (
        lax.dot_general(
            t, w_ref[...], (((1,), (1,)), ((), ())),
            preferred_element_type=jnp.float32,
        )
        + b_ref[...]
    )


def _tc_head(pooled, W, b2d):
    blk = 512
    return pl.pallas_call(
        _tc_head_body,
        grid=(_B // blk,),
        in_specs=[
            pl.BlockSpec((blk, _D), lambda i: (i, 0)),
            pl.BlockSpec((_CLASSES, _D), lambda i: (0, 0)),
            pl.BlockSpec((1, _CLASSES), lambda i: (0, 0)),
        ],
        out_specs=pl.BlockSpec((blk, _CLASSES), lambda i: (i, 0)),
        out_shape=jax.ShapeDtypeStruct((_B, _CLASSES), jnp.float32),
    )(pooled, W, b2d)


@jax.jit
def kernel(x, emb_table, W, b):
    xr = x.reshape(_B * 2, _HALF)
    pooled = _sc_pool(xr, emb_table)
    return _tc_head(pooled, W, b.reshape(1, _CLASSES))


# speedup vs baseline: 2.3627x; 2.3627x over previous
import functools

import jax
import jax.numpy as jnp
from jax import lax
from jax.experimental import pallas as pl
from jax.experimental.pallas import tpu as pltpu
from jax.experimental.pallas import tpu_sc as plsc

_VOCAB = 1000000
_CLASSES = 1000
_D = 32
_B = 4096
_L = 200

_NC = 2
_NS = 16
_NW = _NC * _NS
_ROWS_PER_W = _B // _NW
_HALF = _L // 2
_NBUF = 4


def _sc_pool_body(x_hbm, table_hbm, out_hbm, idx_v, bufs, acc, sems):
    wid = lax.axis_index("s") * _NC + lax.axis_index("c")
    idx_base = wid * (2 * _ROWS_PER_W)
    row_base = wid * _ROWS_PER_W

    pltpu.sync_copy(x_hbm.at[pl.ds(idx_base, 2 * _ROWS_PER_W)], idx_v)

    def fire(b, s):
        pltpu.async_copy(
            table_hbm.at[idx_v.at[2 * b]],
            bufs.at[s, pl.ds(0, _HALF)],
            sems.at[s],
        )
        pltpu.async_copy(
            table_hbm.at[idx_v.at[2 * b + 1]],
            bufs.at[s, pl.ds(_HALF, _HALF)],
            sems.at[s],
        )

    def drain(b, s):
        pltpu.make_async_copy(
            table_hbm.at[idx_v.at[2 * b]],
            bufs.at[s, pl.ds(0, _HALF)],
            sems.at[s],
        ).wait()
        pltpu.make_async_copy(
            table_hbm.at[idx_v.at[2 * b + 1]],
            bufs.at[s, pl.ds(_HALF, _HALF)],
            sems.at[s],
        ).wait()

    for s in range(_NBUF):
        fire(s, s)

    zeros = jnp.zeros((16,), jnp.float32)

    def outer(bb, carry):
        for s in range(_NBUF):
            b = bb * _NBUF + s
            drain(b, s)

            def body(r, c):
                a0, a1, a2, a3 = c
                a0 = a0 + bufs[s, 2 * r, pl.ds(0, 16)]
                a1 = a1 + bufs[s, 2 * r, pl.ds(16, 16)]
                a2 = a2 + bufs[s, 2 * r + 1, pl.ds(0, 16)]
                a3 = a3 + bufs[s, 2 * r + 1, pl.ds(16, 16)]
                return (a0, a1, a2, a3)

            nb = b + _NBUF

            @pl.when(nb < _ROWS_PER_W)
            def _():
                fire(nb, s)

            a0, a1, a2, a3 = lax.fori_loop(
                0, _L // 2, body, (zeros, zeros, zeros, zeros), unroll=2
            )
            acc[b, pl.ds(0, 16)] = a0 + a2
            acc[b, pl.ds(16, 16)] = a1 + a3
        return carry

    lax.fori_loop(0, _ROWS_PER_W // _NBUF, outer, 0)

    pltpu.sync_copy(acc, out_hbm.at[pl.ds(row_base, _ROWS_PER_W)])


@functools.cache
def _sc_pool():
    return pl.kernel(
        _sc_pool_body,
        mesh=plsc.VectorSubcoreMesh(core_axis_name="c", subcore_axis_name="s"),
        compiler_params=pltpu.CompilerParams(use_tc_tiling_on_sc=False),
        out_type=jax.ShapeDtypeStruct((_B, _D), jnp.float32),
        scratch_types=[
            pltpu.VMEM((2 * _ROWS_PER_W, _HALF), jnp.int32),
            pltpu.VMEM((_NBUF, _L, _D), jnp.float32),
            pltpu.VMEM((_ROWS_PER_W, _D), jnp.float32),
            pltpu.SemaphoreType.DMA((_NBUF,)),
        ],
    )


def _tc_head_body(p_ref, w_ref, b_ref, o_ref):
    t = jnp.tanh(p_ref[...] * (1.0 / _L))
    o_ref[...] = (
        lax.dot_general(
            t, w_ref[...], (((1,), (1,)), ((), ())),
            preferred_element_type=jnp.float32,
        )
        + b_ref[...]
    )


def _tc_head(pooled, W, b2d):
    blk = 512
    return pl.pallas_call(
        _tc_head_body,
        grid=(_B // blk,),
        in_specs=[
            pl.BlockSpec((blk, _D), lambda i: (i, 0)),
            pl.BlockSpec((_CLASSES, _D), lambda i: (0, 0)),
            pl.BlockSpec((1, _CLASSES), lambda i: (0, 0)),
        ],
        out_specs=pl.BlockSpec((blk, _CLASSES), lambda i: (i, 0)),
        out_shape=jax.ShapeDtypeStruct((_B, _CLASSES), jnp.float32),
    )(pooled, W, b2d)


@jax.jit
def kernel(x, emb_table, W, b):
    xr = x.reshape(_B * 2, _HALF)
    pooled = _sc_pool()(xr, emb_table)
    return _tc_head(pooled, W, b.reshape(1, _CLASSES))


# speedup vs baseline: 2.3675x; 1.0020x over previous
import functools

import jax
import jax.numpy as jnp
from jax import lax
from jax.experimental import pallas as pl
from jax.experimental.pallas import tpu as pltpu
from jax.experimental.pallas import tpu_sc as plsc

_VOCAB = 1000000
_CLASSES = 1000
_D = 32
_B = 4096
_L = 200

_NC = 2
_NS = 16
_NW = _NC * _NS
_ROWS_PER_W = _B // _NW
_HALF = _L // 2
_NBUF = 4


def _sc_pool_body(x_hbm, table_hbm, out_hbm, idx_v, bufs, acc, sems):
    wid = lax.axis_index("s") * _NC + lax.axis_index("c")
    row_base = wid * _ROWS_PER_W

    pltpu.sync_copy(x_hbm.at[pl.ds(row_base, _ROWS_PER_W)], idx_v)

    def fire(b, s):
        pltpu.async_copy(
            table_hbm.at[idx_v.at[b]],
            bufs.at[s],
            sems.at[s],
        )

    def drain(b, s):
        pltpu.make_async_copy(
            table_hbm.at[idx_v.at[b]],
            bufs.at[s],
            sems.at[s],
        ).wait()

    for s in range(_NBUF):
        fire(s, s)

    zeros = jnp.zeros((16,), jnp.float32)

    def outer(bb, carry):
        for s in range(_NBUF):
            b = bb * _NBUF + s
            drain(b, s)

            def body(r, c):
                a0, a1, a2, a3 = c
                a0 = a0 + bufs[s, 2 * r, pl.ds(0, 16)]
                a1 = a1 + bufs[s, 2 * r, pl.ds(16, 16)]
                a2 = a2 + bufs[s, 2 * r + 1, pl.ds(0, 16)]
                a3 = a3 + bufs[s, 2 * r + 1, pl.ds(16, 16)]
                return (a0, a1, a2, a3)

            nb = b + _NBUF

            @pl.when(nb < _ROWS_PER_W)
            def _():
                fire(nb, s)

            a0, a1, a2, a3 = lax.fori_loop(
                0, _L // 2, body, (zeros, zeros, zeros, zeros), unroll=2
            )
            acc[b, pl.ds(0, 16)] = a0 + a2
            acc[b, pl.ds(16, 16)] = a1 + a3
        return carry

    lax.fori_loop(0, _ROWS_PER_W // _NBUF, outer, 0)

    pltpu.sync_copy(acc, out_hbm.at[pl.ds(row_base, _ROWS_PER_W)])


@functools.cache
def _sc_pool():
    return pl.kernel(
        _sc_pool_body,
        mesh=plsc.VectorSubcoreMesh(core_axis_name="c", subcore_axis_name="s"),
        compiler_params=pltpu.CompilerParams(use_tc_tiling_on_sc=False),
        out_type=jax.ShapeDtypeStruct((_B, _D), jnp.float32),
        scratch_types=[
            pltpu.VMEM((_ROWS_PER_W, _L), jnp.int32),
            pltpu.VMEM((_NBUF, _L, _D), jnp.float32),
            pltpu.VMEM((_ROWS_PER_W, _D), jnp.float32),
            pltpu.SemaphoreType.DMA((_NBUF,)),
        ],
    )


def _tc_head_body(p_ref, w_ref, b_ref, o_ref):
    t = jnp.tanh(p_ref[...] * (1.0 / _L))
    o_ref[...] = (
        lax.dot_general(
            t, w_ref[...], (((1,), (1,)), ((), ())),
            preferred_element_type=jnp.float32,
        )
        + b_ref[...]
    )


def _tc_head(pooled, W, b2d):
    blk = 512
    return pl.pallas_call(
        _tc_head_body,
        grid=(_B // blk,),
        in_specs=[
            pl.BlockSpec((blk, _D), lambda i: (i, 0)),
            pl.BlockSpec((_CLASSES, _D), lambda i: (0, 0)),
            pl.BlockSpec((1, _CLASSES), lambda i: (0, 0)),
        ],
        out_specs=pl.BlockSpec((blk, _CLASSES), lambda i: (i, 0)),
        out_shape=jax.ShapeDtypeStruct((_B, _CLASSES), jnp.float32),
    )(pooled, W, b2d)


@jax.jit
def kernel(x, emb_table, W, b):
    pooled = _sc_pool()(x, emb_table)
    return _tc_head(pooled, W, b.reshape(1, _CLASSES))
